# baseline (speedup 1.0000x reference)
import functools

import jax
import jax.numpy as jnp
from jax.experimental import pallas as pl

_EPS = 1e-5
_B, _N = 4, 4096


def _fold(layer):
    s = layer['gamma'] / jnp.sqrt(1.0 + _EPS)
    return layer['W'] * s[None, :], (layer['b'] * s + layer['beta'])[None, :]


def _mlp_max_kern(x_ref, w1, b1, w2, b2, w3, b3, o_ref, *, gpb, K, cout):
    h = x_ref[...]
    for w, b in ((w1, b1), (w2, b2), (w3, b3)):
        h = jnp.maximum(
            jnp.dot(h, w[...], preferred_element_type=jnp.float32) + b[...], 0.0)
    o_ref[...] = jnp.max(h.reshape(gpb, K, cout), axis=1)


def _mlp_max(x, layers, K, gpb):
    g_total = x.shape[0] // K
    cin = x.shape[1]
    ws_bs = []
    for L in layers:
        w, b = _fold(L)
        ws_bs += [w, b]
    cout = ws_bs[-2].shape[1]
    kern = functools.partial(_mlp_max_kern, gpb=gpb, K=K, cout=cout)
    wspecs = [pl.BlockSpec(a.shape, lambda i: (0, 0)) for a in ws_bs]
    return pl.pallas_call(
        kern,
        grid=(g_total // gpb,),
        in_specs=[pl.BlockSpec((gpb * K, cin), lambda i: (i, 0))] + wspecs,
        out_specs=pl.BlockSpec((gpb, cout), lambda i: (i, 0)),
        out_shape=jax.ShapeDtypeStruct((g_total, cout), jnp.float32),
    )(x, *ws_bs)


def _head_kern(x_ref, w1, b1, w2, b2, w3, b3, o_ref):
    h = x_ref[...]
    h = jnp.maximum(jnp.dot(h, w1[...], preferred_element_type=jnp.float32) + b1[...], 0.0)
    h = jnp.maximum(jnp.dot(h, w2[...], preferred_element_type=jnp.float32) + b2[...], 0.0)
    o_ref[...] = jnp.dot(h, w3[...], preferred_element_type=jnp.float32) + b3[...]


def _head(x, fc1, fc2, fc3):
    w1, b1 = _fold(fc1)
    w2, b2 = _fold(fc2)
    w3, b3 = fc3['W'], fc3['b'][None, :]
    nout = w3.shape[1]
    return pl.pallas_call(
        _head_kern,
        out_shape=jax.ShapeDtypeStruct((x.shape[0], nout), jnp.float32),
    )(x, w1, b1, w2, b2, w3, b3)


def _square_distance(src, dst):
    return (jnp.sum(src ** 2, -1)[:, :, None] + jnp.sum(dst ** 2, -1)[:, None, :]
            - 2.0 * jnp.einsum('bsd,bnd->bsn', src, dst))


def _index_points(points, idx):
    return jax.vmap(lambda p, i: p[i])(points, idx)


def _fps(xyz, npoint):
    b, n, _ = xyz.shape

    def body(i, state):
        centroids, distance, farthest = state
        centroids = centroids.at[:, i].set(farthest)
        centroid = jnp.take_along_axis(xyz, farthest[:, None, None], axis=1)
        dist = jnp.sum((xyz - centroid) ** 2, -1)
        distance = jnp.minimum(distance, dist)
        farthest = jnp.argmax(distance, -1).astype(jnp.int32)
        return centroids, distance, farthest

    init = (jnp.zeros((b, npoint), jnp.int32),
            jnp.full((b, n), 1e10, jnp.float32),
            jnp.zeros((b,), jnp.int32))
    centroids, _, _ = jax.lax.fori_loop(0, npoint, body, init)
    return centroids


def _query_ball(radius, nsample, xyz, new_xyz):
    b, n, _ = xyz.shape
    s = new_xyz.shape[1]
    sqrdists = _square_distance(new_xyz, xyz)
    grp = jnp.broadcast_to(jnp.arange(n, dtype=jnp.int32)[None, None, :], (b, s, n))
    grp = jnp.where(sqrdists > radius ** 2, n, grp)
    grp = jnp.sort(grp, axis=-1)[:, :, :nsample]
    first = jnp.broadcast_to(grp[:, :, :1], grp.shape)
    return jnp.where(grp == n, first, grp)


def _sa_msg(xyz, points, npoint, radius_list, nsample_list, branches, gpb):
    b, n, _ = xyz.shape
    fps_idx = _fps(xyz, npoint)
    new_xyz = _index_points(xyz, fps_idx)
    outs = []
    for radius, K, layers in zip(radius_list, nsample_list, branches):
        idx = _query_ball(radius, K, xyz, new_xyz)
        grouped_xyz = _index_points(xyz, idx) - new_xyz[:, :, None, :]
        if points is None:
            grouped = grouped_xyz
        else:
            grouped = jnp.concatenate([_index_points(points, idx), grouped_xyz], -1)
        cin = grouped.shape[-1]
        flat = grouped.reshape(b * npoint * K, cin)
        out = _mlp_max(flat, layers, K, gpb)
        outs.append(out.reshape(b, npoint, -1))
    return new_xyz, jnp.concatenate(outs, -1)


@jax.jit
def _forward(pointcloud, params):
    xyz = jnp.transpose(pointcloud, (0, 2, 1))
    l1_xyz, l1_points = _sa_msg(xyz, None, 512, [0.1, 0.2, 0.4], [16, 32, 128],
                                params['sa1'], gpb=16)
    l2_xyz, l2_points = _sa_msg(l1_xyz, l1_points, 128, [0.2, 0.4, 0.8],
                                [32, 64, 128], params['sa2'], gpb=16)
    grouped = jnp.concatenate([l2_xyz, l2_points], -1)
    l3 = _mlp_max(grouped.reshape(_B * 128, 643), params['sa3'], 128, 4)
    logits = _head(l3, params['fc1'], params['fc2'], params['fc3'])
    return logits, l3


def kernel(pointcloud, params):
    return _forward(pointcloud, params)


# speedup vs baseline: 1.4526x; 1.4526x over previous
import functools

import jax
import jax.numpy as jnp
from jax.experimental import pallas as pl

_EPS = 1e-5
_B, _N = 4, 4096


def _fold(layer):
    s = layer['gamma'] / jnp.sqrt(1.0 + _EPS)
    return layer['W'] * s[None, :], (layer['b'] * s + layer['beta'])[None, :]


def _mlp_max_kern(x_ref, w1, b1, w2, b2, w3, b3, o_ref, *, gpb, K, cout):
    h = x_ref[...]
    for w, b in ((w1, b1), (w2, b2), (w3, b3)):
        h = jnp.maximum(
            jnp.dot(h, w[...], preferred_element_type=jnp.float32) + b[...], 0.0)
    o_ref[...] = jnp.max(h.reshape(gpb, K, cout), axis=1)


def _mlp_max(x, layers, K, gpb):
    g_total = x.shape[0] // K
    cin = x.shape[1]
    ws_bs = []
    for L in layers:
        w, b = _fold(L)
        ws_bs += [w, b]
    cout = ws_bs[-2].shape[1]
    kern = functools.partial(_mlp_max_kern, gpb=gpb, K=K, cout=cout)
    wspecs = [pl.BlockSpec(a.shape, lambda i: (0, 0)) for a in ws_bs]
    return pl.pallas_call(
        kern,
        grid=(g_total // gpb,),
        in_specs=[pl.BlockSpec((gpb * K, cin), lambda i: (i, 0))] + wspecs,
        out_specs=pl.BlockSpec((gpb, cout), lambda i: (i, 0)),
        out_shape=jax.ShapeDtypeStruct((g_total, cout), jnp.float32),
    )(x, *ws_bs)


def _head_kern(x_ref, w1, b1, w2, b2, w3, b3, o_ref):
    h = x_ref[...]
    h = jnp.maximum(jnp.dot(h, w1[...], preferred_element_type=jnp.float32) + b1[...], 0.0)
    h = jnp.maximum(jnp.dot(h, w2[...], preferred_element_type=jnp.float32) + b2[...], 0.0)
    o_ref[...] = jnp.dot(h, w3[...], preferred_element_type=jnp.float32) + b3[...]


def _head(x, fc1, fc2, fc3):
    w1, b1 = _fold(fc1)
    w2, b2 = _fold(fc2)
    w3, b3 = fc3['W'], fc3['b'][None, :]
    nout = w3.shape[1]
    return pl.pallas_call(
        _head_kern,
        out_shape=jax.ShapeDtypeStruct((x.shape[0], nout), jnp.float32),
    )(x, w1, b1, w2, b2, w3, b3)


def _index_points(points, idx):
    return jax.vmap(lambda p, i: p[i])(points, idx)


def _fps_kern(xt_ref, o_ref, *, npoint, n):
    x = xt_ref[0]
    lane = jax.lax.broadcasted_iota(jnp.int32, (1, n), 1)
    olane = jax.lax.broadcasted_iota(jnp.int32, (1, npoint), 1)

    def body(i, state):
        distance, farthest, cents = state
        cents = jnp.where(olane == i, farthest, cents)
        sel = (lane == farthest).astype(jnp.float32)
        c = jnp.sum(x * sel, axis=1, keepdims=True)
        dist = jnp.sum((x - c) ** 2, axis=0, keepdims=True)
        distance = jnp.minimum(distance, dist)
        m = jnp.max(distance)
        nf = jnp.min(jnp.where(distance == m, lane, n))
        return distance, nf, cents

    init = (jnp.full((1, n), 1e10, jnp.float32), jnp.int32(0),
            jnp.zeros((1, npoint), jnp.int32))
    _, _, cents = jax.lax.fori_loop(0, npoint, body, init)
    o_ref[...] = cents[None]


def _fps(xyz, npoint):
    b, n, _ = xyz.shape
    xt = jnp.transpose(xyz, (0, 2, 1))
    out = pl.pallas_call(
        functools.partial(_fps_kern, npoint=npoint, n=n),
        grid=(b,),
        in_specs=[pl.BlockSpec((1, 3, n), lambda i: (i, 0, 0))],
        out_specs=pl.BlockSpec((1, 1, npoint), lambda i: (i, 0, 0)),
        out_shape=jax.ShapeDtypeStruct((b, 1, npoint), jnp.int32),
    )(xt)
    return out[:, 0]


def _ball_kern(sq_ref, o_ref, *, r2, K, n, sc):
    sq = sq_ref[0]
    mask = sq <= r2
    rank = mask.astype(jnp.int32)
    shift = 1
    while shift < n:
        rank = rank + jnp.pad(rank, ((0, 0), (shift, 0)))[:, :n]
        shift *= 2
    lane = jax.lax.broadcasted_iota(jnp.int32, (sc, n), 1)
    kiota = jax.lax.broadcasted_iota(jnp.int32, (sc, K), 1)
    mrank = jnp.where(mask, rank, 0)

    def body(k, out):
        sel = mrank == k + 1
        idx_k = jnp.sum(jnp.where(sel, lane, 0), axis=1, keepdims=True)
        return jnp.where(kiota == k, idx_k, out)

    out = jax.lax.fori_loop(0, K, body, jnp.zeros((sc, K), jnp.int32))
    count = rank[:, n - 1:n]
    out = jnp.where(kiota < count, out, out[:, 0:1])
    o_ref[...] = jnp.where(count == 0, n, out)[None]


def _query_ball(radius, nsample, sq):
    b, s, n = sq.shape
    sc = min(s, 128)
    return pl.pallas_call(
        functools.partial(_ball_kern, r2=radius ** 2, K=nsample, n=n, sc=sc),
        grid=(b, s // sc),
        in_specs=[pl.BlockSpec((1, sc, n), lambda i, j: (i, j, 0))],
        out_specs=pl.BlockSpec((1, sc, nsample), lambda i, j: (i, j, 0)),
        out_shape=jax.ShapeDtypeStruct((b, s, nsample), jnp.int32),
    )(sq)


def _sa_msg(xyz, points, npoint, radius_list, nsample_list, branches, gpb):
    b, n, _ = xyz.shape
    fps_idx = _fps(xyz, npoint)
    new_xyz = _index_points(xyz, fps_idx)
    sq = (jnp.sum(new_xyz ** 2, -1)[:, :, None] + jnp.sum(xyz ** 2, -1)[:, None, :]
          - 2.0 * jnp.einsum('bsd,bnd->bsn', new_xyz, xyz))
    outs = []
    for radius, K, layers in zip(radius_list, nsample_list, branches):
        idx = _query_ball(radius, K, sq)
        grouped_xyz = _index_points(xyz, idx) - new_xyz[:, :, None, :]
        if points is None:
            grouped = grouped_xyz
        else:
            grouped = jnp.concatenate([_index_points(points, idx), grouped_xyz], -1)
        cin = grouped.shape[-1]
        flat = grouped.reshape(b * npoint * K, cin)
        out = _mlp_max(flat, layers, K, gpb)
        outs.append(out.reshape(b, npoint, -1))
    return new_xyz, jnp.concatenate(outs, -1)


@jax.jit
def _forward(pointcloud, params):
    xyz = jnp.transpose(pointcloud, (0, 2, 1))
    l1_xyz, l1_points = _sa_msg(xyz, None, 512, [0.1, 0.2, 0.4], [16, 32, 128],
                                params['sa1'], gpb=16)
    l2_xyz, l2_points = _sa_msg(l1_xyz, l1_points, 128, [0.2, 0.4, 0.8],
                                [32, 64, 128], params['sa2'], gpb=16)
    grouped = jnp.concatenate([l2_xyz, l2_points], -1)
    l3 = _mlp_max(grouped.reshape(_B * 128, 643), params['sa3'], 128, 4)
    logits = _head(l3, params['fc1'], params['fc2'], params['fc3'])
    return logits, l3


def kernel(pointcloud, params):
    return _forward(pointcloud, params)
